# Initial kernel scaffold; baseline (speedup 1.0000x reference)
import jax
import jax.numpy as jnp
from jax.experimental import pallas as pl


def kernel(x, edge_index, node_to_subgraph, W1a, b1a, W2a, b2a, W1b, b1b, W2b, b2b, W1c, b1c, W2c, b2c):
    raise NotImplementedError("write your pallas kernel here")


# speedup vs baseline: 13.9856x; 13.9856x over previous
import jax
import jax.numpy as jnp
from jax import lax
from jax.experimental import pallas as pl
from jax.experimental.pallas import tpu as pltpu
from jax.experimental.pallas import tpu_sc as plsc

N = 100000
E = 3200000
H = 64
S = 10000

NC = 2
NS = 16
G = 4
FW = 16
CH = 128
RB = 6

NT = 102400
SP = 10240
CHT = 25344
EPAD = CHT * CH
CPT = CHT // NS
CPW = CHT // (NC * NS)
PCT = NT // CH
PCPT = PCT // NS
PRB = 10

_f32 = jnp.float32


def _edge_sweep(tbl, src2d, dst2d, acc, sidx, didx, rows, gsem, ssem,
                chunk0, nchunks):
    def body(ob, carry):
        base = chunk0 + ob * (2 * RB)
        pltpu.sync_copy(src2d.at[pl.ds(base, RB)], sidx.at[0])
        pltpu.sync_copy(dst2d.at[pl.ds(base, RB)], didx.at[0])
        ga = [pltpu.async_copy(tbl.at[sidx.at[0, j]], rows.at[0, j], gsem)
              for j in range(RB)]
        pltpu.sync_copy(src2d.at[pl.ds(base + RB, RB)], sidx.at[1])
        pltpu.sync_copy(dst2d.at[pl.ds(base + RB, RB)], didx.at[1])
        gb = [pltpu.async_copy(tbl.at[sidx.at[1, j]], rows.at[1, j], gsem)
              for j in range(RB)]
        for d in ga:
            d.wait()
        sa = [pltpu.async_copy(rows.at[0, j], acc.at[didx.at[0, j]], ssem,
                               add=True) for j in range(RB)]
        for d in gb:
            d.wait()
        sb = [pltpu.async_copy(rows.at[1, j], acc.at[didx.at[1, j]], ssem,
                               add=True) for j in range(RB)]
        for d in sa:
            d.wait()
        for d in sb:
            d.wait()
        return carry

    lax.fori_loop(0, nchunks // (2 * RB), body, 0)


def _agg_bc_body(h_hbm, src2d, dst2d, out_hbm,
                 sidx, didx, rows, acc, gsem, ssem):
    c = lax.axis_index("c")
    s = lax.axis_index("s")
    r0 = s * (NT // NS)
    for gl in range(G // NC):
        g = c * (G // NC) + gl
        pltpu.sync_copy(h_hbm.at[g, pl.ds(r0, NT // NS)],
                        acc.at[pl.ds(r0, NT // NS)])
        plsc.subcore_barrier()
        _edge_sweep(h_hbm.at[g], src2d, dst2d, acc, sidx, didx, rows,
                    gsem, ssem, s * CPT, CPT)
        plsc.subcore_barrier()
        pltpu.sync_copy(acc.at[pl.ds(r0, NT // NS)],
                        out_hbm.at[g, pl.ds(r0, NT // NS)])
        plsc.subcore_barrier()


def _agg_a_body(x16, src2d, dst2d, zn, out_hbm,
                sidx, didx, rows, acc, gsem, ssem):
    c = lax.axis_index("c")
    s = lax.axis_index("s")
    wid = c * NS + s
    r0 = s * (NT // NS)
    pltpu.sync_copy(zn.at[pl.ds(r0, NT // NS)], acc.at[pl.ds(r0, NT // NS)])
    plsc.subcore_barrier()
    _edge_sweep(x16, src2d, dst2d, acc, sidx, didx, rows, gsem, ssem,
                wid * CPW, CPW)
    plsc.subcore_barrier()
    pltpu.sync_copy(acc.at[pl.ds(r0, NT // NS)],
                    out_hbm.at[c, pl.ds(r0, NT // NS)])
    plsc.subcore_barrier()


def _pool_body(h_hbm, nts2d, zsp, out_hbm, ridx, rowsp, accp, ssem):
    c = lax.axis_index("c")
    s = lax.axis_index("s")
    p0 = s * (SP // NS)
    for gl in range(G // NC):
        g = c * (G // NC) + gl
        pltpu.sync_copy(zsp.at[pl.ds(p0, SP // NS)],
                        accp.at[pl.ds(p0, SP // NS)])
        plsc.subcore_barrier()

        def body(ob, carry):
            cb = s * PCPT + ob * PRB
            pltpu.sync_copy(h_hbm.at[g, pl.ds(cb * CH, PRB * CH)], rowsp)
            pltpu.sync_copy(nts2d.at[pl.ds(cb, PRB)], ridx)
            sds = [pltpu.async_copy(rowsp.at[pl.ds(j * CH, CH)],
                                    accp.at[ridx.at[j]], ssem, add=True)
                   for j in range(PRB)]
            for d in sds:
                d.wait()
            return carry

        lax.fori_loop(0, PCPT // PRB, body, 0)
        plsc.subcore_barrier()
        pltpu.sync_copy(accp.at[pl.ds(p0, SP // NS)],
                        out_hbm.at[g, pl.ds(p0, SP // NS)])
        plsc.subcore_barrier()


_sc_mesh = plsc.VectorSubcoreMesh(core_axis_name="c", subcore_axis_name="s")
_sc_params = pltpu.CompilerParams(use_tc_tiling_on_sc=False)

_agg_bc = pl.kernel(
    _agg_bc_body,
    out_type=jax.ShapeDtypeStruct((G, NT, FW), _f32),
    mesh=_sc_mesh,
    compiler_params=_sc_params,
    scratch_types=[
        pltpu.VMEM((2, RB, CH), jnp.int32),
        pltpu.VMEM((2, RB, CH), jnp.int32),
        pltpu.VMEM((2, RB, CH, FW), _f32),
        pltpu.VMEM_SHARED((NT, FW), _f32),
        pltpu.SemaphoreType.DMA,
        pltpu.SemaphoreType.DMA,
    ],
)

_agg_a = pl.kernel(
    _agg_a_body,
    out_type=jax.ShapeDtypeStruct((NC, NT, FW), _f32),
    mesh=_sc_mesh,
    compiler_params=_sc_params,
    scratch_types=[
        pltpu.VMEM((2, RB, CH), jnp.int32),
        pltpu.VMEM((2, RB, CH), jnp.int32),
        pltpu.VMEM((2, RB, CH, FW), _f32),
        pltpu.VMEM_SHARED((NT, FW), _f32),
        pltpu.SemaphoreType.DMA,
        pltpu.SemaphoreType.DMA,
    ],
)

_pool = pl.kernel(
    _pool_body,
    out_type=jax.ShapeDtypeStruct((G, SP, FW), _f32),
    mesh=_sc_mesh,
    compiler_params=_sc_params,
    scratch_types=[
        pltpu.VMEM((PRB, CH), jnp.int32),
        pltpu.VMEM((PRB * CH, FW), _f32),
        pltpu.VMEM_SHARED((SP, FW), _f32),
        pltpu.SemaphoreType.DMA,
    ],
)

BN = 2048
NBLK = NT // BN


def _mlp_a_body(a2_ref, x_ref, w1, b1, w2, b2, o_ref):
    sval = a2_ref[0, :, 0:1] + a2_ref[1, :, 0:1] + x_ref[...]
    h1 = jnp.maximum(sval * w1[...] + b1[...], 0.0)
    h2 = jnp.maximum(
        jnp.dot(h1, w2[...], preferred_element_type=_f32) + b2[...], 0.0)
    for gi in range(G):
        o_ref[gi] = h2[:, gi * FW:(gi + 1) * FW]


def _mlp_bc_body(g_ref, w1, b1, w2, b2, o_ref):
    acc = jnp.dot(g_ref[0], w1[0:FW, :], preferred_element_type=_f32)
    for gi in range(1, G):
        acc = acc + jnp.dot(g_ref[gi], w1[gi * FW:(gi + 1) * FW, :],
                            preferred_element_type=_f32)
    h1 = jnp.maximum(acc + b1[...], 0.0)
    h2 = jnp.maximum(
        jnp.dot(h1, w2[...], preferred_element_type=_f32) + b2[...], 0.0)
    for gi in range(G):
        o_ref[gi] = h2[:, gi * FW:(gi + 1) * FW]


_W_SPEC = pl.BlockSpec((H, H), lambda i: (0, 0))
_B_SPEC = pl.BlockSpec((1, H), lambda i: (0, 0))
_G_SPEC = pl.BlockSpec((G, BN, FW), lambda i: (0, i, 0))

_mlp_a = pl.pallas_call(
    _mlp_a_body,
    grid=(NBLK,),
    in_specs=[
        pl.BlockSpec((NC, BN, FW), lambda i: (0, i, 0)),
        pl.BlockSpec((BN, 1), lambda i: (i, 0)),
        pl.BlockSpec((1, H), lambda i: (0, 0)),
        _B_SPEC, _W_SPEC, _B_SPEC,
    ],
    out_specs=_G_SPEC,
    out_shape=jax.ShapeDtypeStruct((G, NT, FW), _f32),
)

_mlp_bc = pl.pallas_call(
    _mlp_bc_body,
    grid=(NBLK,),
    in_specs=[_G_SPEC, _W_SPEC, _B_SPEC, _W_SPEC, _B_SPEC],
    out_specs=_G_SPEC,
    out_shape=jax.ShapeDtypeStruct((G, NT, FW), _f32),
)


def kernel(x, edge_index, node_to_subgraph,
           W1a, b1a, W2a, b2a,
           W1b, b1b, W2b, b2b,
           W1c, b1c, W2c, b2c):
    src = edge_index[0]
    dst = edge_index[1]
    epad = jnp.full((EPAD - E,), N, dtype=jnp.int32)
    src2d = jnp.concatenate([src, epad]).reshape(CHT, CH)
    dst2d = jnp.concatenate([dst, epad]).reshape(CHT, CH)
    xp = jnp.pad(x, ((0, NT - N), (0, 0)))
    x16 = jnp.broadcast_to(xp, (NT, FW))
    zn = jnp.zeros((NT, FW), _f32)
    zsp = jnp.zeros((SP, FW), _f32)
    nts2d = jnp.concatenate(
        [node_to_subgraph, jnp.full((NT - N,), S, jnp.int32)]).reshape(PCT, CH)

    a2 = _agg_a(x16, src2d, dst2d, zn)
    h1 = _mlp_a(a2, xp, W1a, b1a.reshape(1, H), W2a, b2a.reshape(1, H))
    g1 = _agg_bc(h1, src2d, dst2d)
    h2 = _mlp_bc(g1, W1b, b1b.reshape(1, H), W2b, b2b.reshape(1, H))
    g2 = _agg_bc(h2, src2d, dst2d)
    h3 = _mlp_bc(g2, W1c, b1c.reshape(1, H), W2c, b2c.reshape(1, H))
    p = _pool(h3, nts2d, zsp)
    out = p[:, :S, :].transpose(1, 0, 2).reshape(S, H)
    return out
